# Initial kernel scaffold; baseline (speedup 1.0000x reference)
import jax
import jax.numpy as jnp
from jax.experimental import pallas as pl


def kernel(x, s0_w1, s0_g1, s0_b1, s0_w2, s0_g2, s0_b2, s1_w1, s1_g1, s1_b1, s1_w2, s1_g2, s1_b2, s2_w1, s2_g1, s2_b1, s2_w2, s2_g2, s2_b2, s3_w1, s3_g1, s3_b1, s3_w2, s3_g2, s3_b2, s4_w1, s4_g1, s4_b1, s4_w2, s4_g2, s4_b2):
    raise NotImplementedError("write your pallas kernel here")


# speedup vs baseline: 3.0840x; 3.0840x over previous
import jax
import jax.numpy as jnp
from jax import lax
from jax.experimental import pallas as pl
from jax.experimental.pallas import tpu as pltpu

EPS = 1e-5
F32 = jnp.float32


def _conv(x, wmat, sc, sh, H, W, CIN, COUT, TH):
    n = x.shape[0]
    half = n // 2

    def body(x_ref, w_ref, sc_ref, sh_ref, y_ref):
        xt = x_ref[0].reshape(H, W, CIN)
        z = jnp.maximum(xt * sc_ref[0] + sh_ref[0], 0.0)
        cz = jnp.zeros((H, 1, CIN), F32)
        rz = jnp.zeros((1, W + 2, CIN), F32)
        zp = jnp.concatenate(
            [rz, jnp.concatenate([cz, z, cz], axis=1), rz], axis=0)
        w = w_ref[...]
        for r0 in range(0, H, TH):
            cols = [zp[r0 + dy:r0 + dy + TH, dx:dx + W, :]
                    for dy in range(3) for dx in range(3)]
            p = jnp.concatenate(cols, axis=-1).reshape(TH * W, 9 * CIN)
            acc = jnp.dot(p, w, preferred_element_type=F32)
            y_ref[0, r0 * W:(r0 + TH) * W, :] = acc

    return pl.pallas_call(
        body,
        out_shape=jax.ShapeDtypeStruct((n, H * W, COUT), F32),
        grid_spec=pltpu.PrefetchScalarGridSpec(
            num_scalar_prefetch=0,
            grid=(2, half),
            in_specs=[
                pl.BlockSpec((1, H * W, CIN),
                             lambda i, j: (i * half + j, 0, 0)),
                pl.BlockSpec((9 * CIN, COUT), lambda i, j: (0, 0)),
                pl.BlockSpec((1, CIN), lambda i, j: (0, 0)),
                pl.BlockSpec((1, CIN), lambda i, j: (0, 0)),
            ],
            out_specs=pl.BlockSpec((1, H * W, COUT),
                                   lambda i, j: (i * half + j, 0, 0))),
        compiler_params=pltpu.CompilerParams(
            dimension_semantics=("parallel", "arbitrary"),
            vmem_limit_bytes=64 * 1024 * 1024),
        cost_estimate=pl.CostEstimate(
            flops=2 * n * H * W * 9 * CIN * COUT,
            transcendentals=0,
            bytes_accessed=4 * n * H * W * (CIN + COUT)),
    )(x, wmat, sc, sh)


def _s0_conv(patches, wmat, m, COUT):
    n = patches.shape[0]
    half = n // 2

    def body(p_ref, w_ref, y_ref):
        w = w_ref[...]
        for r0 in range(0, m, 256):
            y_ref[0, r0:r0 + 256, :] = jnp.dot(
                p_ref[0, r0:r0 + 256, :], w, preferred_element_type=F32)

    return pl.pallas_call(
        body,
        out_shape=jax.ShapeDtypeStruct((n, m, COUT), F32),
        grid_spec=pltpu.PrefetchScalarGridSpec(
            num_scalar_prefetch=0,
            grid=(2, half),
            in_specs=[
                pl.BlockSpec((1, m, 16), lambda i, j: (i * half + j, 0, 0)),
                pl.BlockSpec((16, COUT), lambda i, j: (0, 0)),
            ],
            out_specs=pl.BlockSpec((1, m, COUT),
                                   lambda i, j: (i * half + j, 0, 0))),
        compiler_params=pltpu.CompilerParams(
            dimension_semantics=("parallel", "arbitrary"),
            vmem_limit_bytes=64 * 1024 * 1024),
    )(patches, wmat)


def _stats(y, C):
    m = y.shape[0]

    def body(y_ref, st_ref):
        @pl.when(pl.program_id(0) == 0)
        def _():
            st_ref[...] = jnp.zeros_like(st_ref)

        t = y_ref[...]
        st_ref[...] += jnp.concatenate(
            [jnp.sum(t, axis=0, keepdims=True),
             jnp.sum(t * t, axis=0, keepdims=True)], axis=0)

    return pl.pallas_call(
        body,
        out_shape=jax.ShapeDtypeStruct((2, C), F32),
        grid_spec=pltpu.PrefetchScalarGridSpec(
            num_scalar_prefetch=0,
            grid=(m // 256,),
            in_specs=[pl.BlockSpec((256, C), lambda t: (t, 0))],
            out_specs=pl.BlockSpec((2, C), lambda t: (0, 0))),
        compiler_params=pltpu.CompilerParams(
            dimension_semantics=("arbitrary",)),
    )(y)


def _pool(y, sc, sh, H, W, C):
    n = y.shape[0]
    ho, wo = H // 2, W // 2

    def body(y_ref, sc_ref, sh_ref, o_ref):
        z = y_ref[0].reshape(H, W, C)
        z = jnp.maximum(z * sc_ref[0] + sh_ref[0], 0.0)
        p = z.reshape(ho, 2, wo, 2, C).max(axis=(1, 3))
        o_ref[0] = p.reshape(ho * wo, C)

    return pl.pallas_call(
        body,
        out_shape=jax.ShapeDtypeStruct((n, ho * wo, C), F32),
        grid_spec=pltpu.PrefetchScalarGridSpec(
            num_scalar_prefetch=0,
            grid=(n,),
            in_specs=[
                pl.BlockSpec((1, H * W, C), lambda i: (i, 0, 0)),
                pl.BlockSpec((1, C), lambda i: (0, 0)),
                pl.BlockSpec((1, C), lambda i: (0, 0)),
            ],
            out_specs=pl.BlockSpec((1, ho * wo, C), lambda i: (i, 0, 0))),
        compiler_params=pltpu.CompilerParams(
            dimension_semantics=("parallel",)),
    )(y, sc, sh)


def _final(y, sc, sh, H, W, C):
    n = y.shape[0]

    def body(y_ref, sc_ref, sh_ref, o_ref):
        z = jnp.maximum(y_ref[0] * sc_ref[0] + sh_ref[0], 0.0)
        o_ref[0] = z.reshape(H, W, C)

    return pl.pallas_call(
        body,
        out_shape=jax.ShapeDtypeStruct((n, H, W, C), F32),
        grid_spec=pltpu.PrefetchScalarGridSpec(
            num_scalar_prefetch=0,
            grid=(n,),
            in_specs=[
                pl.BlockSpec((1, H * W, C), lambda i: (i, 0, 0)),
                pl.BlockSpec((1, C), lambda i: (0, 0)),
                pl.BlockSpec((1, C), lambda i: (0, 0)),
            ],
            out_specs=pl.BlockSpec((1, H, W, C), lambda i: (i, 0, 0, 0))),
        compiler_params=pltpu.CompilerParams(
            dimension_semantics=("parallel",)),
    )(y, sc, sh)


def _affine(y, g, b, C):
    st = _stats(y.reshape(-1, C), C)
    m = y.shape[0] * y.shape[1]
    mean = st[0] / m
    var = jnp.maximum(st[1] / m - mean * mean, 0.0)
    sc = g * lax.rsqrt(var + EPS)
    sh = b - mean * sc
    return sc[None, :], sh[None, :]


def kernel(x, s0_w1, s0_g1, s0_b1, s0_w2, s0_g2, s0_b2,
           s1_w1, s1_g1, s1_b1, s1_w2, s1_g2, s1_b2,
           s2_w1, s2_g1, s2_b1, s2_w2, s2_g2, s2_b2,
           s3_w1, s3_g1, s3_b1, s3_w2, s3_g2, s3_b2,
           s4_w1, s4_g1, s4_b1, s4_w2, s4_g2, s4_b2):
    n = x.shape[0]

    x2 = x.reshape(n, 256, 256)
    xp = x2.reshape(n, 128, 2, 128, 2).max(axis=(2, 4))
    xpp = jnp.pad(xp, ((0, 0), (1, 1), (1, 1)))
    taps = [xpp[:, dy:dy + 128, dx:dx + 128]
            for dy in range(3) for dx in range(3)]
    patches = jnp.stack(taps, axis=-1)
    patches = jnp.pad(patches, ((0, 0), (0, 0), (0, 0), (0, 7)))
    patches = patches.reshape(n, 128 * 128, 16)
    w0 = jnp.pad(s0_w1.reshape(9, 64), ((0, 7), (0, 0)))

    plan = [
        (128, 128, 64, 64, 2),
        (64, 64, 64, 64, 4),
        (32, 32, 64, 128, 8),
        (16, 16, 128, 128, 16),
        (8, 8, 128, 128, 8),
    ]
    params = [
        (s0_w1, s0_g1, s0_b1, s0_w2, s0_g2, s0_b2),
        (s1_w1, s1_g1, s1_b1, s1_w2, s1_g2, s1_b2),
        (s2_w1, s2_g1, s2_b1, s2_w2, s2_g2, s2_b2),
        (s3_w1, s3_g1, s3_b1, s3_w2, s3_g2, s3_b2),
        (s4_w1, s4_g1, s4_b1, s4_w2, s4_g2, s4_b2),
    ]

    y = _s0_conv(patches, w0, 128 * 128, 64)
    sc, sh = _affine(y, s0_g1, s0_b1, 64)

    for si, (H, W, CIN, COUT, TH) in enumerate(plan):
        w1, g1, b1, w2, g2, b2 = params[si]
        if si > 0:
            ones = jnp.ones((1, CIN), F32)
            zeros = jnp.zeros((1, CIN), F32)
            y = _conv(y, w1.reshape(9 * CIN, COUT), ones, zeros,
                      H, W, CIN, COUT, TH)
            sc, sh = _affine(y, g1, b1, COUT)
        y = _conv(y, w2.reshape(9 * COUT, COUT), sc, sh,
                  H, W, COUT, COUT, TH)
        sc, sh = _affine(y, g2, b2, COUT)
        if si < 4:
            y = _pool(y, sc, sh, H, W, COUT)

    out = _final(y, sc, sh, 8, 8, 128)
    return jnp.transpose(out, (0, 3, 1, 2))
